# baseline (speedup 1.0000x reference)
import jax
import jax.numpy as jnp
import numpy as np
from jax import lax
from jax.experimental import pallas as pl
from jax.experimental.pallas import tpu as pltpu
from jax.experimental.pallas import tpu_sc as plsc

NC = 2
NS = 16
C = 128
D = 4
NW = NC * NS


def _sc_mesh():
    return plsc.VectorSubcoreMesh(
        core_axis_name="c", subcore_axis_name="s", num_cores=NC, num_subcores=NS
    )


def _wid_and_count(nch):
    c = lax.axis_index("c")
    s = lax.axis_index("s")
    wid = s * NC + c
    kk, r = nch // NW, nch % NW
    kt = jnp.where(wid < r, kk + 1, kk).astype(jnp.int32)
    return c, s, wid, kt


def _idx_ring(ei_hbm, ring, sem, wid, plane_off):
    def start(j, m):
        g = plane_off + (wid + NW * j) * C
        pltpu.async_copy(ei_hbm.at[pl.ds(g, C)], ring.at[m], sem.at[m])

    def wait(j, m):
        g = plane_off + (wid + NW * j) * C
        pltpu.make_async_copy(
            ei_hbm.at[pl.ds(g, C)], ring.at[m], sem.at[m]).wait()

    return start, wait


def _deg_kernel(n_pad, nch):
    sl = n_pad // NS

    def body(ei_hbm, ones_hbm, zeros_hbm, degp_hbm,
             cring, ones_v, accum, isem, ssem):
        c, s, wid, kt = _wid_and_count(nch)
        base = s * sl
        pltpu.sync_copy(zeros_hbm.at[pl.ds(base, sl)], accum.at[pl.ds(base, sl)])
        pltpu.sync_copy(ones_hbm, ones_v)
        plsc.subcore_barrier()

        i_start, i_wait = _idx_ring(ei_hbm, cring, isem, wid, nch * C)

        def s_start(j, b):
            pltpu.async_copy(
                ones_v, accum.at[cring.at[lax.rem(j, D)]], ssem.at[b],
                add=True)

        def s_wait(j, b):
            pltpu.make_async_copy(
                ones_v, accum.at[cring.at[lax.rem(j, D)]], ssem.at[b]).wait()

        for d in range(D):
            i_start(d, d)

        def step(j, carry):
            b = lax.rem(j, 2)
            i_wait(j, lax.rem(j, D))
            s_start(j, b)

            @pl.when(j >= 1)
            def _():
                s_wait(j - 1, 1 - b)

                @pl.when(j - 1 + D < kt)
                def _():
                    i_start(j - 1 + D, lax.rem(j - 1, D))

            return carry

        lax.fori_loop(0, kt, step, 0)
        s_wait(kt - 1, lax.rem(kt - 1, 2))
        plsc.subcore_barrier()
        pltpu.sync_copy(accum.at[pl.ds(base, sl)], degp_hbm.at[c, pl.ds(base, sl)])

    return pl.kernel(
        body,
        out_type=jax.ShapeDtypeStruct((NC, n_pad), jnp.float32),
        mesh=_sc_mesh(),
        scratch_types=[
            pltpu.VMEM((D, C), jnp.int32),
            pltpu.VMEM((C,), jnp.float32),
            pltpu.VMEM_SHARED((n_pad,), jnp.float32),
            pltpu.SemaphoreType.DMA((D,)),
            pltpu.SemaphoreType.DMA((2,)),
        ],
    )


def _edge_gs_kernel(n_pad, nch, width):
    sl = n_pad // NS

    def body(tab_hbm, ei_hbm, zeros_hbm, part0_hbm, part1_hbm,
             rring, cring, gbuf, accum, rsem, csem, gsem, ssem):
        c, s, wid, kt = _wid_and_count(nch)
        base = s * sl
        pltpu.sync_copy(zeros_hbm.at[pl.ds(base, sl)], accum.at[pl.ds(base, sl)])
        plsc.subcore_barrier()

        r_start, r_wait = _idx_ring(ei_hbm, rring, rsem, wid, 0)
        c_start, c_wait = _idx_ring(ei_hbm, cring, csem, wid, nch * C)

        def i_start(j, m):
            r_start(j, m)
            c_start(j, m)

        def i_wait(j, m):
            r_wait(j, m)
            c_wait(j, m)

        def g_start(j, b):
            pltpu.async_copy(
                tab_hbm.at[rring.at[lax.rem(j, D)]], gbuf.at[b], gsem.at[b])

        def g_wait(j, b):
            pltpu.make_async_copy(
                tab_hbm.at[rring.at[lax.rem(j, D)]], gbuf.at[b],
                gsem.at[b]).wait()

        def s_start(j, b):
            pltpu.async_copy(
                gbuf.at[b], accum.at[cring.at[lax.rem(j, D)]], ssem.at[b],
                add=True)

        def s_wait(j, b):
            pltpu.make_async_copy(
                gbuf.at[b], accum.at[cring.at[lax.rem(j, D)]],
                ssem.at[b]).wait()

        for d in range(D):
            i_start(d, d)
        i_wait(0, 0)
        g_start(0, 0)

        def step(j, carry):
            b = lax.rem(j, 2)
            g_wait(j, b)
            s_start(j, b)

            @pl.when(j >= 1)
            def _():
                s_wait(j - 1, 1 - b)
                @pl.when(j - 1 + D < kt)
                def _():
                    i_start(j - 1 + D, lax.rem(j - 1, D))

            @pl.when(j + 1 < kt)
            def _():
                i_wait(j + 1, lax.rem(j + 1, D))
                g_start(j + 1, 1 - b)

            return carry

        lax.fori_loop(0, kt, step, 0)
        s_wait(kt - 1, lax.rem(kt - 1, 2))
        plsc.subcore_barrier()

        @pl.when(c == 0)
        def _():
            pltpu.sync_copy(accum.at[pl.ds(base, sl)], part0_hbm.at[pl.ds(base, sl)])

        @pl.when(c == 1)
        def _():
            pltpu.sync_copy(accum.at[pl.ds(base, sl)], part1_hbm.at[pl.ds(base, sl)])

    return pl.kernel(
        body,
        out_type=(
            jax.ShapeDtypeStruct((n_pad, width), jnp.float32),
            jax.ShapeDtypeStruct((n_pad, width), jnp.float32),
        ),
        mesh=_sc_mesh(),
        scratch_types=[
            pltpu.VMEM((D, C), jnp.int32),
            pltpu.VMEM((D, C), jnp.int32),
            pltpu.VMEM((2, C, width), jnp.float32),
            pltpu.VMEM_SHARED((n_pad, width), jnp.float32),
            pltpu.SemaphoreType.DMA((D,)),
            pltpu.SemaphoreType.DMA((D,)),
            pltpu.SemaphoreType.DMA((2,)),
            pltpu.SemaphoreType.DMA((2,)),
        ],
    )


def _edge_gs1_kernel(n_pad, nch):
    sl = n_pad // NS

    def body(z_hbm, ei_hbm, zeros_hbm, part_hbm,
             z_s, rring, cring, gbuf, accum, rsem, csem, gsem, ssem):
        c, s, wid, kt = _wid_and_count(nch)
        base = s * sl
        pltpu.sync_copy(zeros_hbm.at[pl.ds(base, sl)], accum.at[pl.ds(base, sl)])
        pltpu.sync_copy(z_hbm.at[pl.ds(base, sl)], z_s.at[pl.ds(base, sl)])
        plsc.subcore_barrier()

        r_start, r_wait = _idx_ring(ei_hbm, rring, rsem, wid, 0)
        c_start, c_wait = _idx_ring(ei_hbm, cring, csem, wid, nch * C)

        def i_start(j, m):
            r_start(j, m)
            c_start(j, m)

        def i_wait(j, m):
            r_wait(j, m)
            c_wait(j, m)

        def g_start(j, b):
            pltpu.async_copy(
                z_s.at[rring.at[lax.rem(j, D)]], gbuf.at[b], gsem.at[b])

        def g_wait(j, b):
            pltpu.make_async_copy(
                z_s.at[rring.at[lax.rem(j, D)]], gbuf.at[b],
                gsem.at[b]).wait()

        def s_start(j, b):
            pltpu.async_copy(
                gbuf.at[b], accum.at[cring.at[lax.rem(j, D)]], ssem.at[b],
                add=True)

        def s_wait(j, b):
            pltpu.make_async_copy(
                gbuf.at[b], accum.at[cring.at[lax.rem(j, D)]],
                ssem.at[b]).wait()

        for d in range(D):
            i_start(d, d)
        i_wait(0, 0)
        g_start(0, 0)

        def step(j, carry):
            b = lax.rem(j, 2)
            g_wait(j, b)
            s_start(j, b)

            @pl.when(j >= 1)
            def _():
                s_wait(j - 1, 1 - b)

                @pl.when(j - 1 + D < kt)
                def _():
                    i_start(j - 1 + D, lax.rem(j - 1, D))

            @pl.when(j + 1 < kt)
            def _():
                i_wait(j + 1, lax.rem(j + 1, D))
                g_start(j + 1, 1 - b)

            return carry

        lax.fori_loop(0, kt, step, 0)
        s_wait(kt - 1, lax.rem(kt - 1, 2))
        plsc.subcore_barrier()
        pltpu.sync_copy(accum.at[pl.ds(base, sl)], part_hbm.at[c, pl.ds(base, sl)])

    return pl.kernel(
        body,
        out_type=jax.ShapeDtypeStruct((NC, n_pad), jnp.float32),
        mesh=_sc_mesh(),
        scratch_types=[
            pltpu.VMEM_SHARED((n_pad,), jnp.float32),
            pltpu.VMEM((D, C), jnp.int32),
            pltpu.VMEM((D, C), jnp.int32),
            pltpu.VMEM((2, C), jnp.float32),
            pltpu.VMEM_SHARED((n_pad,), jnp.float32),
            pltpu.SemaphoreType.DMA((D,)),
            pltpu.SemaphoreType.DMA((D,)),
            pltpu.SemaphoreType.DMA((2,)),
            pltpu.SemaphoreType.DMA((2,)),
        ],
    )


def _tc1(xp_ref, w1t_ref, degp_ref, y_ref, dis_ref):
    deg = degp_ref[0] + degp_ref[1] + 1.0
    dis = lax.rsqrt(deg)
    xw = jnp.dot(xp_ref[...], w1t_ref[...], preferred_element_type=jnp.float32)
    y_ref[...] = xw * dis
    dis_ref[...] = dis


def _tc2(part0_ref, part1_ref, y_ref, dis_ref, b1_ref, w4t_ref, z_ref):
    seg = part0_ref[...] + part1_ref[...] + y_ref[...]
    h = jnp.maximum(seg * dis_ref[...] + b1_ref[...], 0.0)
    hw = jnp.dot(h, w4t_ref[...], preferred_element_type=jnp.float32)
    z_ref[...] = hw * dis_ref[...]


def _tc3(part2_ref, z_ref, dis_ref, b4_ref, out_ref):
    o = (part2_ref[0] + part2_ref[1] + z_ref[...]) * dis_ref[...] + b4_ref[...]
    out_ref[...] = jax.nn.sigmoid(o)


def kernel(x, edge_index, W1, b1, W4, b4):
    n, f = x.shape
    e = edge_index.shape[1]
    h = W1.shape[0]

    assert e % C == 0, "edge count must split into 128-edge chunks"
    nch = e // C
    n_pad = -(-n // (NS * 128)) * NS * 128
    f_pad = -(-f // 8) * 8

    ei32 = edge_index.astype(jnp.int32).reshape(2 * e)

    xp = jnp.pad(x, ((0, n_pad - n), (0, f_pad - f)))
    w1t = jnp.pad(W1, ((0, 0), (0, f_pad - f))).T
    w4t = W4.T
    zeros1 = np.zeros((n_pad,), np.float32)
    zeros2 = np.zeros((n_pad, h), np.float32)
    ones_c = np.ones((C,), np.float32)

    degp = _deg_kernel(n_pad, nch)(ei32, ones_c, zeros1)

    y, dis = pl.pallas_call(
        _tc1,
        out_shape=(
            jax.ShapeDtypeStruct((n_pad, h), jnp.float32),
            jax.ShapeDtypeStruct((n_pad, 1), jnp.float32),
        ),
    )(xp, w1t, degp.reshape(NC, n_pad, 1))

    part0, part1 = _edge_gs_kernel(n_pad, nch, h)(y, ei32, zeros2)

    z = pl.pallas_call(
        _tc2,
        out_shape=jax.ShapeDtypeStruct((n_pad, 1), jnp.float32),
    )(part0, part1, y, dis, b1.reshape(1, h), w4t)

    part2 = _edge_gs1_kernel(n_pad, nch)(z.reshape(n_pad), ei32, zeros1)

    out = pl.pallas_call(
        _tc3,
        out_shape=jax.ShapeDtypeStruct((n_pad, 1), jnp.float32),
    )(part2.reshape(NC, n_pad, 1), z, dis, b4.reshape(1, 1))

    return out[:n]


# speedup vs baseline: 1.0019x; 1.0019x over previous
import jax
import jax.numpy as jnp
import numpy as np
from jax import lax
from jax.experimental import pallas as pl
from jax.experimental.pallas import tpu as pltpu
from jax.experimental.pallas import tpu_sc as plsc

NC = 2
NS = 16
C = 128
D = 4
NW = NC * NS


def _sc_mesh():
    return plsc.VectorSubcoreMesh(
        core_axis_name="c", subcore_axis_name="s", num_cores=NC, num_subcores=NS
    )


def _wid_and_count(nch):
    c = lax.axis_index("c")
    s = lax.axis_index("s")
    wid = s * NC + c
    kk, r = nch // NW, nch % NW
    kt = jnp.where(wid < r, kk + 1, kk).astype(jnp.int32)
    return c, s, wid, kt


def _idx_ring(ei_hbm, ring, sem, wid, plane):
    def start(j, m):
        g = (wid + NW * j) * C
        pltpu.async_copy(ei_hbm.at[:, pl.ds(g, C)], ring.at[m], sem.at[m])

    def wait(j, m):
        g = (wid + NW * j) * C
        pltpu.make_async_copy(
            ei_hbm.at[:, pl.ds(g, C)], ring.at[m], sem.at[m]).wait()

    return start, wait


def _deg_kernel(n_pad, nch):
    sl = n_pad // NS

    def body(ei_hbm, ones_hbm, zeros_hbm, degp_hbm,
             ring, ones_v, accum, isem, ssem):
        c, s, wid, kt = _wid_and_count(nch)
        base = s * sl
        pltpu.sync_copy(zeros_hbm.at[pl.ds(base, sl)], accum.at[pl.ds(base, sl)])
        pltpu.sync_copy(ones_hbm, ones_v)
        plsc.subcore_barrier()

        i_start, i_wait = _idx_ring(ei_hbm, ring, isem, wid, 1)

        def s_start(j, b):
            pltpu.async_copy(
                ones_v, accum.at[ring.at[lax.rem(j, D), 1]], ssem.at[b],
                add=True)

        def s_wait(j, b):
            pltpu.make_async_copy(
                ones_v, accum.at[ring.at[lax.rem(j, D), 1]], ssem.at[b]).wait()

        for d in range(D):
            i_start(d, d)

        def step(j, carry):
            b = lax.rem(j, 2)
            i_wait(j, lax.rem(j, D))
            s_start(j, b)

            @pl.when(j >= 1)
            def _():
                s_wait(j - 1, 1 - b)

                @pl.when(j - 1 + D < kt)
                def _():
                    i_start(j - 1 + D, lax.rem(j - 1, D))

            return carry

        lax.fori_loop(0, kt, step, 0)
        s_wait(kt - 1, lax.rem(kt - 1, 2))
        plsc.subcore_barrier()
        pltpu.sync_copy(accum.at[pl.ds(base, sl)], degp_hbm.at[c, pl.ds(base, sl)])

    return pl.kernel(
        body,
        out_type=jax.ShapeDtypeStruct((NC, n_pad), jnp.float32),
        mesh=_sc_mesh(),
        compiler_params=pltpu.CompilerParams(use_tc_tiling_on_sc=True),
        scratch_types=[
            pltpu.VMEM((D, 2, C), jnp.int32),
            pltpu.VMEM((C,), jnp.float32),
            pltpu.VMEM_SHARED((n_pad,), jnp.float32),
            pltpu.SemaphoreType.DMA((D,)),
            pltpu.SemaphoreType.DMA((2,)),
        ],
    )


def _edge_gs_kernel(n_pad, nch, width):
    sl = n_pad // NS

    def body(tab_hbm, ei_hbm, zeros_hbm, part0_hbm, part1_hbm,
             ring, gbuf, accum, isem, gsem, ssem):
        c, s, wid, kt = _wid_and_count(nch)
        base = s * sl
        pltpu.sync_copy(zeros_hbm.at[pl.ds(base, sl)], accum.at[pl.ds(base, sl)])
        plsc.subcore_barrier()

        i_start, i_wait = _idx_ring(ei_hbm, ring, isem, wid, 0)

        def g_start(j, b):
            pltpu.async_copy(
                tab_hbm.at[ring.at[lax.rem(j, D), 0]], gbuf.at[b], gsem.at[b])

        def g_wait(j, b):
            pltpu.make_async_copy(
                tab_hbm.at[ring.at[lax.rem(j, D), 0]], gbuf.at[b],
                gsem.at[b]).wait()

        def s_start(j, b):
            pltpu.async_copy(
                gbuf.at[b], accum.at[ring.at[lax.rem(j, D), 1]], ssem.at[b],
                add=True)

        def s_wait(j, b):
            pltpu.make_async_copy(
                gbuf.at[b], accum.at[ring.at[lax.rem(j, D), 1]],
                ssem.at[b]).wait()

        for d in range(D):
            i_start(d, d)
        i_wait(0, 0)
        g_start(0, 0)

        def step(j, carry):
            b = lax.rem(j, 2)
            g_wait(j, b)
            s_start(j, b)

            @pl.when(j >= 1)
            def _():
                s_wait(j - 1, 1 - b)
                @pl.when(j - 1 + D < kt)
                def _():
                    i_start(j - 1 + D, lax.rem(j - 1, D))

            @pl.when(j + 1 < kt)
            def _():
                i_wait(j + 1, lax.rem(j + 1, D))
                g_start(j + 1, 1 - b)

            return carry

        lax.fori_loop(0, kt, step, 0)
        s_wait(kt - 1, lax.rem(kt - 1, 2))
        plsc.subcore_barrier()

        @pl.when(c == 0)
        def _():
            pltpu.sync_copy(accum.at[pl.ds(base, sl)], part0_hbm.at[pl.ds(base, sl)])

        @pl.when(c == 1)
        def _():
            pltpu.sync_copy(accum.at[pl.ds(base, sl)], part1_hbm.at[pl.ds(base, sl)])

    return pl.kernel(
        body,
        out_type=(
            jax.ShapeDtypeStruct((n_pad, width), jnp.float32),
            jax.ShapeDtypeStruct((n_pad, width), jnp.float32),
        ),
        mesh=_sc_mesh(),
        compiler_params=pltpu.CompilerParams(use_tc_tiling_on_sc=True),
        scratch_types=[
            pltpu.VMEM((D, 2, C), jnp.int32),
            pltpu.VMEM((2, C, width), jnp.float32),
            pltpu.VMEM_SHARED((n_pad, width), jnp.float32),
            pltpu.SemaphoreType.DMA((D,)),
            pltpu.SemaphoreType.DMA((2,)),
            pltpu.SemaphoreType.DMA((2,)),
        ],
    )


def _edge_gs1_kernel(n_pad, nch):
    sl = n_pad // NS

    def body(z_hbm, ei_hbm, zeros_hbm, part_hbm,
             z_s, ring, gbuf, accum, isem, gsem, ssem):
        c, s, wid, kt = _wid_and_count(nch)
        base = s * sl
        pltpu.sync_copy(zeros_hbm.at[pl.ds(base, sl)], accum.at[pl.ds(base, sl)])
        pltpu.sync_copy(z_hbm.at[pl.ds(base, sl)], z_s.at[pl.ds(base, sl)])
        plsc.subcore_barrier()

        i_start, i_wait = _idx_ring(ei_hbm, ring, isem, wid, 0)

        def g_start(j, b):
            pltpu.async_copy(
                z_s.at[ring.at[lax.rem(j, D), 0]], gbuf.at[b], gsem.at[b])

        def g_wait(j, b):
            pltpu.make_async_copy(
                z_s.at[ring.at[lax.rem(j, D), 0]], gbuf.at[b],
                gsem.at[b]).wait()

        def s_start(j, b):
            pltpu.async_copy(
                gbuf.at[b], accum.at[ring.at[lax.rem(j, D), 1]], ssem.at[b],
                add=True)

        def s_wait(j, b):
            pltpu.make_async_copy(
                gbuf.at[b], accum.at[ring.at[lax.rem(j, D), 1]],
                ssem.at[b]).wait()

        for d in range(D):
            i_start(d, d)
        i_wait(0, 0)
        g_start(0, 0)

        def step(j, carry):
            b = lax.rem(j, 2)
            g_wait(j, b)
            s_start(j, b)

            @pl.when(j >= 1)
            def _():
                s_wait(j - 1, 1 - b)

                @pl.when(j - 1 + D < kt)
                def _():
                    i_start(j - 1 + D, lax.rem(j - 1, D))

            @pl.when(j + 1 < kt)
            def _():
                i_wait(j + 1, lax.rem(j + 1, D))
                g_start(j + 1, 1 - b)

            return carry

        lax.fori_loop(0, kt, step, 0)
        s_wait(kt - 1, lax.rem(kt - 1, 2))
        plsc.subcore_barrier()
        pltpu.sync_copy(accum.at[pl.ds(base, sl)], part_hbm.at[c, pl.ds(base, sl)])

    return pl.kernel(
        body,
        out_type=jax.ShapeDtypeStruct((NC, n_pad), jnp.float32),
        mesh=_sc_mesh(),
        compiler_params=pltpu.CompilerParams(use_tc_tiling_on_sc=True),
        scratch_types=[
            pltpu.VMEM_SHARED((n_pad,), jnp.float32),
            pltpu.VMEM((D, 2, C), jnp.int32),
            pltpu.VMEM((2, C), jnp.float32),
            pltpu.VMEM_SHARED((n_pad,), jnp.float32),
            pltpu.SemaphoreType.DMA((D,)),
            pltpu.SemaphoreType.DMA((2,)),
            pltpu.SemaphoreType.DMA((2,)),
        ],
    )


def _tc1(xp_ref, w1t_ref, degp_ref, y_ref, dis_ref):
    deg = degp_ref[0] + degp_ref[1] + 1.0
    dis = lax.rsqrt(deg)
    xw = jnp.dot(xp_ref[...], w1t_ref[...], preferred_element_type=jnp.float32)
    y_ref[...] = xw * dis
    dis_ref[...] = dis


def _tc2(part0_ref, part1_ref, y_ref, dis_ref, b1_ref, w4t_ref, z_ref):
    seg = part0_ref[...] + part1_ref[...] + y_ref[...]
    h = jnp.maximum(seg * dis_ref[...] + b1_ref[...], 0.0)
    hw = jnp.dot(h, w4t_ref[...], preferred_element_type=jnp.float32)
    z_ref[...] = hw * dis_ref[...]


def _tc3(part2_ref, z_ref, dis_ref, b4_ref, out_ref):
    o = (part2_ref[0] + part2_ref[1] + z_ref[...]) * dis_ref[...] + b4_ref[...]
    out_ref[...] = jax.nn.sigmoid(o)


def kernel(x, edge_index, W1, b1, W4, b4):
    n, f = x.shape
    e = edge_index.shape[1]
    h = W1.shape[0]

    assert e % C == 0, "edge count must split into 128-edge chunks"
    nch = e // C
    n_pad = -(-n // (NS * 128)) * NS * 128
    f_pad = -(-f // 8) * 8

    ei32 = edge_index.astype(jnp.int32)

    xp = jnp.pad(x, ((0, n_pad - n), (0, f_pad - f)))
    w1t = jnp.pad(W1, ((0, 0), (0, f_pad - f))).T
    w4t = W4.T
    zeros1 = np.zeros((n_pad,), np.float32)
    zeros2 = np.zeros((n_pad, h), np.float32)
    ones_c = np.ones((C,), np.float32)

    degp = _deg_kernel(n_pad, nch)(ei32, ones_c, zeros1)

    y, dis = pl.pallas_call(
        _tc1,
        out_shape=(
            jax.ShapeDtypeStruct((n_pad, h), jnp.float32),
            jax.ShapeDtypeStruct((n_pad, 1), jnp.float32),
        ),
    )(xp, w1t, degp.reshape(NC, n_pad, 1))

    part0, part1 = _edge_gs_kernel(n_pad, nch, h)(y, ei32, zeros2)

    z = pl.pallas_call(
        _tc2,
        out_shape=jax.ShapeDtypeStruct((n_pad, 1), jnp.float32),
    )(part0, part1, y, dis, b1.reshape(1, h), w4t)

    part2 = _edge_gs1_kernel(n_pad, nch)(z.reshape(n_pad), ei32, zeros1)

    out = pl.pallas_call(
        _tc3,
        out_shape=jax.ShapeDtypeStruct((n_pad, 1), jnp.float32),
    )(part2.reshape(NC, n_pad, 1), z, dis, b4.reshape(1, 1))

    return out[:n]
